# baseline (speedup 1.0000x reference)
import functools

import jax
import jax.numpy as jnp
from jax import lax
from jax.experimental import pallas as pl
from jax.experimental.pallas import tpu as pltpu
from jax.experimental.pallas import tpu_sc as plsc

_C = 768
_HW = 3136
_NP = 3584
_BQ = 1792
_BK = 3584
_NQB = _NP // _BQ
_NKB = _NP // _BK

_NW = 32
_BPW = _NP // _NW


def _argmin_body(ct_ref, tm_ref, z_ref, bs_ref):
    q = pl.program_id(0)

    a = ct_ref[...]

    @pl.when(q == 0)
    def _():
        bf = tm_ref[...].astype(jnp.float32)
        bn2 = jnp.sum(bf * bf, axis=0, keepdims=True)
        bs_ref[...] = (bf / (jnp.sqrt(bn2 + 1e-8) + 1e-8)).astype(jnp.bfloat16)

    r = lax.dot_general(a, bs_ref[...], (((1,), (0,)), ((), ())),
                        preferred_element_type=jnp.float32)
    sim = r[:, :_HW]
    m_blk = jnp.max(sim, axis=1, keepdims=True)
    col = lax.broadcasted_iota(jnp.int32, (_BQ, _HW), 1)
    z_ref[0] = jnp.min(jnp.where(sim == m_blk, col, jnp.int32(2**30)),
                       axis=1, keepdims=True)


@functools.cache
def _sc_gather_fn():
    mesh = plsc.VectorSubcoreMesh(core_axis_name="c", subcore_axis_name="s")

    @functools.partial(
        pl.kernel,
        mesh=mesh,
        out_type=jax.ShapeDtypeStruct((_NP, _C), jnp.float32),
        scratch_types=[
            pltpu.VMEM((_BPW,), jnp.int32),
            pltpu.VMEM((_BPW, _C), jnp.float32),
            pltpu.SemaphoreType.DMA,
            pltpu.SemaphoreType.DMA,
        ],
    )
    def _sc_gather(s_hbm, idx_hbm, out_hbm, idx_v, rows_v, gsem, wsem):
        wid = lax.axis_index("s") * 2 + lax.axis_index("c")
        base = wid * _BPW
        half = _BPW // 2
        pltpu.sync_copy(idx_hbm.at[pl.ds(base, _BPW)], idx_v)
        g0 = pltpu.async_copy(s_hbm.at[idx_v.at[pl.ds(0, half)]],
                              rows_v.at[pl.ds(0, half)], gsem)
        g1 = pltpu.async_copy(s_hbm.at[idx_v.at[pl.ds(half, half)]],
                              rows_v.at[pl.ds(half, half)], gsem)
        g0.wait()
        w0 = pltpu.async_copy(rows_v.at[pl.ds(0, half)],
                              out_hbm.at[pl.ds(base, half)], wsem)
        g1.wait()
        w1 = pltpu.async_copy(rows_v.at[pl.ds(half, half)],
                              out_hbm.at[pl.ds(base + half, half)], wsem)
        w0.wait()
        w1.wait()

    return _sc_gather


def _loss_body(ct_ref, g_ref, out_ref, acc_ref):
    q = pl.program_id(0)

    a = ct_ref[...].astype(jnp.float32)
    g = g_ref[...]
    an2 = jnp.sum(a * a, axis=1, keepdims=True)
    gn2 = jnp.sum(g * g, axis=1, keepdims=True)
    dots = jnp.sum(a * g, axis=1, keepdims=True)
    p = dots / ((jnp.sqrt(an2) + 1e-8) * (jnp.sqrt(gn2) + 1e-8))

    @pl.when(q == 0)
    def _():
        acc_ref[0, 0] = 0.0

    acc_ref[0, 0] += jnp.sum(p)

    @pl.when(q == _NQB - 1)
    def _():
        out_ref[0, 0] = 1.0 - acc_ref[0, 0] / _HW


def kernel(ct_feats, tmpl_feats, s_feats):
    n, c, h, w = ct_feats.shape
    hw = h * w
    pad = _NP - hw
    a_bf = jnp.pad(ct_feats.reshape(c, hw).T,
                   ((0, pad), (0, 0))).astype(jnp.bfloat16)
    b_bf = jnp.pad(tmpl_feats.reshape(c, hw),
                   ((0, 0), (0, pad))).astype(jnp.bfloat16)
    s_t = s_feats.reshape(c, hw).T

    z = pl.pallas_call(
        _argmin_body,
        grid=(_NQB,),
        in_specs=[
            pl.BlockSpec((_BQ, _C), lambda q: (q, 0)),
            pl.BlockSpec((_C, _NP), lambda q: (0, 0)),
        ],
        out_specs=pl.BlockSpec((1, _BQ, 1), lambda q: (q, 0, 0)),
        out_shape=jax.ShapeDtypeStruct((_NQB, _BQ, 1), jnp.int32),
        scratch_shapes=[
            pltpu.VMEM((_C, _NP), jnp.bfloat16),
        ],
    )(a_bf, b_bf)

    feat = _sc_gather_fn()(s_t, z.reshape(_NP))

    out = pl.pallas_call(
        _loss_body,
        grid=(_NQB,),
        in_specs=[
            pl.BlockSpec((_BQ, _C), lambda q: (q, 0)),
            pl.BlockSpec((_BQ, _C), lambda q: (q, 0)),
        ],
        out_specs=pl.BlockSpec(memory_space=pltpu.SMEM),
        out_shape=jax.ShapeDtypeStruct((1, 1), jnp.float32),
        scratch_shapes=[pltpu.SMEM((1, 1), jnp.float32)],
    )(a_bf, feat)
    return out[0, 0]


# speedup vs baseline: 1.0133x; 1.0133x over previous
import functools

import jax
import jax.numpy as jnp
from jax import lax
from jax.experimental import pallas as pl
from jax.experimental.pallas import tpu as pltpu
from jax.experimental.pallas import tpu_sc as plsc

_C = 768
_HW = 3136
_NP = 3584
_BQ = 896
_NQB = _NP // _BQ

_NW = 32
_BPW = _NP // _NW


def _argmin_body(ct_ref, tm_ref, z_ref, bs_ref):
    q = pl.program_id(0)

    a = ct_ref[...]

    @pl.when(q == 0)
    def _():
        bf = tm_ref[...].astype(jnp.float32)
        bn2 = jnp.sum(bf * bf, axis=0, keepdims=True)
        bs_ref[...] = (bf / (jnp.sqrt(bn2 + 1e-8) + 1e-8)).astype(jnp.bfloat16)

    r = lax.dot_general(a, bs_ref[...], (((1,), (0,)), ((), ())),
                        preferred_element_type=jnp.float32)
    sim = r[:, :_HW]
    m_blk = jnp.max(sim, axis=1, keepdims=True)
    col = lax.broadcasted_iota(jnp.int32, (_BQ, _HW), 1)
    z_ref[0] = jnp.min(jnp.where(sim == m_blk, col, jnp.int32(2**30)),
                       axis=1, keepdims=True)


@functools.cache
def _sc_gather_fn():
    mesh = plsc.VectorSubcoreMesh(core_axis_name="c", subcore_axis_name="s")

    @functools.partial(
        pl.kernel,
        mesh=mesh,
        out_type=jax.ShapeDtypeStruct((_NP, _C), jnp.float32),
        scratch_types=[
            pltpu.VMEM((_BPW,), jnp.int32),
            pltpu.VMEM((_BPW, _C), jnp.float32),
            pltpu.SemaphoreType.DMA,
            pltpu.SemaphoreType.DMA,
        ],
    )
    def _sc_gather(s_hbm, idx_hbm, out_hbm, idx_v, rows_v, gsem, wsem):
        wid = lax.axis_index("s") * 2 + lax.axis_index("c")
        base = wid * _BPW
        half = _BPW // 2
        pltpu.sync_copy(idx_hbm.at[pl.ds(base, _BPW)], idx_v)
        g0 = pltpu.async_copy(s_hbm.at[idx_v.at[pl.ds(0, half)]],
                              rows_v.at[pl.ds(0, half)], gsem)
        g1 = pltpu.async_copy(s_hbm.at[idx_v.at[pl.ds(half, half)]],
                              rows_v.at[pl.ds(half, half)], gsem)
        g0.wait()
        w0 = pltpu.async_copy(rows_v.at[pl.ds(0, half)],
                              out_hbm.at[pl.ds(base, half)], wsem)
        g1.wait()
        w1 = pltpu.async_copy(rows_v.at[pl.ds(half, half)],
                              out_hbm.at[pl.ds(base + half, half)], wsem)
        w0.wait()
        w1.wait()

    return _sc_gather


def _loss_body(ct_ref, g_ref, out_ref, acc_ref):
    q = pl.program_id(0)

    a = ct_ref[...].astype(jnp.float32)
    g = g_ref[...]
    an2 = jnp.sum(a * a, axis=1, keepdims=True)
    gn2 = jnp.sum(g * g, axis=1, keepdims=True)
    dots = jnp.sum(a * g, axis=1, keepdims=True)
    p = dots / ((jnp.sqrt(an2) + 1e-8) * (jnp.sqrt(gn2) + 1e-8))

    @pl.when(q == 0)
    def _():
        acc_ref[0, 0] = 0.0

    acc_ref[0, 0] += jnp.sum(p)

    @pl.when(q == _NQB - 1)
    def _():
        out_ref[0, 0] = 1.0 - acc_ref[0, 0] / _HW


def kernel(ct_feats, tmpl_feats, s_feats):
    n, c, h, w = ct_feats.shape
    hw = h * w
    pad = _NP - hw
    a_bf = jnp.pad(ct_feats.reshape(c, hw).T,
                   ((0, pad), (0, 0))).astype(jnp.bfloat16)
    b_bf = jnp.pad(tmpl_feats.reshape(c, hw),
                   ((0, 0), (0, pad))).astype(jnp.bfloat16)
    s_t = s_feats.reshape(c, hw).T

    z = pl.pallas_call(
        _argmin_body,
        grid=(_NQB,),
        in_specs=[
            pl.BlockSpec((_BQ, _C), lambda q: (q, 0)),
            pl.BlockSpec((_C, _NP), lambda q: (0, 0)),
        ],
        out_specs=pl.BlockSpec((1, _BQ, 1), lambda q: (q, 0, 0)),
        out_shape=jax.ShapeDtypeStruct((_NQB, _BQ, 1), jnp.int32),
        scratch_shapes=[
            pltpu.VMEM((_C, _NP), jnp.bfloat16),
        ],
    )(a_bf, b_bf)

    feat = _sc_gather_fn()(s_t, z.reshape(_NP))

    out = pl.pallas_call(
        _loss_body,
        grid=(_NQB,),
        in_specs=[
            pl.BlockSpec((_BQ, _C), lambda q: (q, 0)),
            pl.BlockSpec((_BQ, _C), lambda q: (q, 0)),
        ],
        out_specs=pl.BlockSpec(memory_space=pltpu.SMEM),
        out_shape=jax.ShapeDtypeStruct((1, 1), jnp.float32),
        scratch_shapes=[pltpu.SMEM((1, 1), jnp.float32)],
    )(a_bf, feat)
    return out[0, 0]
